# baseline (speedup 1.0000x reference)
import functools

import jax
import jax.numpy as jnp
from jax import lax
from jax.experimental import pallas as pl
from jax.experimental.pallas import tpu as pltpu
from jax.experimental.pallas import tpu_sc as plsc

N = 10000
E = 320000
D = 128
OUT_DIM = 64
NB = 16

NC = 2
NS = 16
EPT = E // (NC * NS)
C = 40
NCHUNK = EPT // C
CZ = 200
NCZ = N // CZ
NBUF = 4


def _sc_conv(h, src, dst, attr, We, be):
    mesh = plsc.VectorSubcoreMesh(core_axis_name="c", subcore_axis_name="s")

    @functools.partial(
        pl.kernel,
        mesh=mesh,
        out_type=jax.ShapeDtypeStruct((NC, N, D), jnp.float32),
        scratch_types=[
            pltpu.VMEM((NBUF, C), jnp.int32),
            pltpu.VMEM((NBUF, C), jnp.int32),
            [pltpu.VMEM((C, D), jnp.float32) for _ in range(NBUF)],
            [pltpu.VMEM((C * 4,), jnp.float32) for _ in range(NBUF)],
            pltpu.VMEM((4, D), jnp.float32),
            pltpu.VMEM((D,), jnp.float32),
            pltpu.VMEM_SHARED((N, D), jnp.float32),
            [pltpu.SemaphoreType.DMA for _ in range(NBUF)],
            [pltpu.SemaphoreType.DMA for _ in range(NBUF)],
            [pltpu.SemaphoreType.DMA for _ in range(NBUF)],
            [pltpu.SemaphoreType.DMA for _ in range(NBUF)],
            [pltpu.SemaphoreType.DMA for _ in range(NBUF)],
        ],
    )
    def body(h_hbm, src_hbm, dst_hbm, attr_hbm, we_hbm, be_hbm, out_hbm,
             idx_s, idx_d, rows, abuf, we_v, be_v, accum,
             isem, dsem, gsem, asem, ssem):
        cid = lax.axis_index("c")
        sid = lax.axis_index("s")
        ebase = (sid * NC + cid) * EPT

        pltpu.sync_copy(we_hbm, we_v)
        pltpu.sync_copy(be_hbm, be_v)

        def zrow(r, _):
            for c8 in range(D // 16):
                rows[NBUF - 1][r, pl.ds(c8 * 16, 16)] = (
                    jnp.zeros((16,), jnp.float32))
            return 0
        lax.fori_loop(0, C, zrow, 0)
        for t in range(N // C // NS + 1):
            zj = sid + NS * t

            @pl.when(zj < N // C)
            def _z():
                pltpu.sync_copy(rows[NBUF - 1], accum.at[pl.ds(zj * C, C)])

        def issue_idx(j, b):
            pltpu.async_copy(src_hbm.at[pl.ds(ebase + j * C, C)],
                             idx_s.at[b], isem[b])
            pltpu.async_copy(dst_hbm.at[pl.ds(ebase + j * C, C)],
                             idx_d.at[b], dsem[b])

        def issue_gather(j, b):
            pltpu.make_async_copy(src_hbm.at[pl.ds(0, C)], idx_s.at[b],
                                  isem[b]).wait()
            pltpu.async_copy(h_hbm.at[idx_s.at[b]], rows[b], gsem[b])
            pltpu.async_copy(attr_hbm.at[pl.ds((ebase + j * C) * 4, C * 4)],
                             abuf[b], asem[b])

        def wait_scatter(b):
            pltpu.make_async_copy(rows[b], accum.at[pl.ds(0, C)],
                                  ssem[b]).wait()

        def compute(j, b):
            pltpu.make_async_copy(h_hbm.at[pl.ds(0, C)], rows[b],
                                  gsem[b]).wait()
            pltpu.make_async_copy(attr_hbm.at[pl.ds(0, C * 4)], abuf[b],
                                  asem[b]).wait()

            for p in range(D // 32):
                sa, sb = pl.ds(2 * p * 16, 16), pl.ds((2 * p + 1) * 16, 16)
                wa = [we_v[k, sa] for k in range(4)]
                wb = [we_v[k, sb] for k in range(4)]
                ba = be_v[sa]
                bb = be_v[sb]

                @plsc.parallel_loop(0, C // 4, 1, unroll=4)
                def _grp(g):
                    av = abuf[b][pl.ds(g * 16, 16)]
                    for i in range(4):
                        r = g * 4 + i
                        sp = [jnp.full((16,), av[4 * i + k]) for k in range(4)]
                        acca = rows[b][r, sa] + ba
                        accb = rows[b][r, sb] + bb
                        for k in range(4):
                            acca = acca + sp[k] * wa[k]
                            accb = accb + sp[k] * wb[k]
                        rows[b][r, sa] = jnp.maximum(acca, 0.0)
                        rows[b][r, sb] = jnp.maximum(accb, 0.0)

            pltpu.make_async_copy(dst_hbm.at[pl.ds(0, C)], idx_d.at[b],
                                  dsem[b]).wait()
            pltpu.async_copy(rows[b], accum.at[idx_d.at[b]], ssem[b],
                             add=True)

        def step(j, b, issue_g=True, wait_s=True, issue_i=True,
                 guard_issue_i=False):
            if issue_g:
                issue_gather(j + 1, (b + 1) % NBUF)
            compute(j, b)
            b3 = (b + 3) % NBUF
            if wait_s:
                wait_scatter(b3)
            if issue_i:
                if guard_issue_i:
                    @pl.when(j + 3 < NCHUNK)
                    def _gi():
                        issue_idx(j + 3, b3)
                else:
                    issue_idx(j + 3, b3)

        issue_idx(0, 0)
        issue_idx(1, 1)
        issue_idx(2, 2)
        issue_gather(0, 0)
        plsc.subcore_barrier()

        step(0, 0, wait_s=False)
        step(1, 1)
        step(2, 2)
        step(3, 3)

        def group(g, _):
            for b in range(NBUF):
                step(NBUF * g + b, b, guard_issue_i=True)
            return 0
        lax.fori_loop(1, (NCHUNK - 2) // NBUF, group, 0)

        step(NCHUNK - 2, (NCHUNK - 2) % NBUF, issue_i=False)
        step(NCHUNK - 1, (NCHUNK - 1) % NBUF, issue_g=False, issue_i=False)
        wait_scatter((NCHUNK - 1) % NBUF)
        plsc.subcore_barrier()

        for t in range((NCZ + NS - 1) // NS):
            oj = sid + NS * t

            @pl.when(oj < NCZ)
            def _w():
                pltpu.sync_copy(accum.at[pl.ds(oj * CZ, CZ)],
                                out_hbm.at[cid, pl.ds(oj * CZ, CZ)])

    return body(h, src, dst, attr.reshape(E * 4), We, be)


_RB = 1000


def _linin_body(x_ref, w1_ref, b1_ref, w2_ref, b2_ref, o_ref):
    h = jnp.maximum(
        jnp.dot(x_ref[...], w1_ref[...], preferred_element_type=jnp.float32)
        + b1_ref[...], 0.0)
    o_ref[...] = jnp.maximum(
        jnp.dot(h, w2_ref[...], preferred_element_type=jnp.float32)
        + b2_ref[...], 0.0)


def _linin(x, W1, b1, W2, b2):
    grid = N // _RB
    return pl.pallas_call(
        _linin_body,
        grid=(grid,),
        in_specs=[
            pl.BlockSpec((_RB, D), lambda i: (i, 0)),
            pl.BlockSpec((D, D), lambda i: (0, 0)),
            pl.BlockSpec((1, D), lambda i: (0, 0)),
            pl.BlockSpec((D, D), lambda i: (0, 0)),
            pl.BlockSpec((1, D), lambda i: (0, 0)),
        ],
        out_specs=pl.BlockSpec((_RB, D), lambda i: (i, 0)),
        out_shape=jax.ShapeDtypeStruct((N, D), jnp.float32),
    )(x, W1, b1.reshape(1, D), W2, b2.reshape(1, D))


def _post_body(p_ref, w_ref, b_ref, o_ref):
    agg = p_ref[0] + p_ref[1]
    o_ref[...] = jnp.maximum(
        jnp.dot(agg, w_ref[...], preferred_element_type=jnp.float32)
        + b_ref[...], 0.0)


def _post(p, Wn, bn):
    grid = N // _RB
    return pl.pallas_call(
        _post_body,
        grid=(grid,),
        in_specs=[
            pl.BlockSpec((NC, _RB, D), lambda i: (0, i, 0)),
            pl.BlockSpec((D, D), lambda i: (0, 0)),
            pl.BlockSpec((1, D), lambda i: (0, 0)),
        ],
        out_specs=pl.BlockSpec((_RB, D), lambda i: (i, 0)),
        out_shape=jax.ShapeDtypeStruct((N, D), jnp.float32),
    )(p, Wn, bn.reshape(1, D))


def _final_body(p_ref, w_ref, b_ref, bt_ref, wp_ref, bp_ref, o_ref,
                sums_ref, cnts_ref):
    i = pl.program_id(0)

    @pl.when(i == 0)
    def _init():
        sums_ref[...] = jnp.zeros_like(sums_ref)
        cnts_ref[...] = jnp.zeros_like(cnts_ref)

    agg = p_ref[0] + p_ref[1]
    h2 = jnp.maximum(
        jnp.dot(agg, w_ref[...], preferred_element_type=jnp.float32)
        + b_ref[...], 0.0)
    bt = bt_ref[0]
    onehot = (jnp.broadcast_to(bt, (NB, _RB))
              == lax.broadcasted_iota(jnp.int32, (NB, _RB), 0)
              ).astype(jnp.float32)
    sums_ref[...] += lax.dot_general(
        onehot, h2, (((1,), (0,)), ((), ())),
        preferred_element_type=jnp.float32)
    cnts_ref[...] += lax.dot_general(
        onehot, jnp.ones((_RB, D), jnp.float32), (((1,), (0,)), ((), ())),
        preferred_element_type=jnp.float32)

    @pl.when(i == pl.num_programs(0) - 1)
    def _fin():
        hg = sums_ref[...] / jnp.maximum(cnts_ref[...], 1.0)
        o_ref[...] = (jnp.dot(hg, wp_ref[...],
                              preferred_element_type=jnp.float32)
                      + bp_ref[...])


def _final(p, Wn, bn, batch3d, Wp, bp):
    grid = N // _RB
    return pl.pallas_call(
        _final_body,
        grid=(grid,),
        in_specs=[
            pl.BlockSpec((NC, _RB, D), lambda i: (0, i, 0)),
            pl.BlockSpec((D, D), lambda i: (0, 0)),
            pl.BlockSpec((1, D), lambda i: (0, 0)),
            pl.BlockSpec((1, 1, _RB), lambda i: (i, 0, 0)),
            pl.BlockSpec((D, OUT_DIM), lambda i: (0, 0)),
            pl.BlockSpec((1, OUT_DIM), lambda i: (0, 0)),
        ],
        out_specs=pl.BlockSpec((NB, OUT_DIM), lambda i: (0, 0)),
        out_shape=jax.ShapeDtypeStruct((NB, OUT_DIM), jnp.float32),
        scratch_shapes=[
            pltpu.VMEM((NB, D), jnp.float32),
            pltpu.VMEM((NB, D), jnp.float32),
        ],
    )(p, Wn, bn.reshape(1, D), batch3d, Wp, bp.reshape(1, OUT_DIM))


def kernel(x, edge_index, edge_attr, batch, W_in1, b_in1, W_in2, b_in2,
           We0, be0, Wn0, bn0, We1, be1, Wn1, bn1, Wp, bp):
    src = edge_index[0]
    dst = edge_index[1]
    batch3d = batch.reshape(N // _RB, 1, _RB)

    h = _linin(x, W_in1, b_in1, W_in2, b_in2)
    p = _sc_conv(h, src, dst, edge_attr, We0, be0)
    h = _post(p, Wn0, bn0)
    p = _sc_conv(h, src, dst, edge_attr, We1, be1)
    return _final(p, Wn1, bn1, batch3d, Wp, bp)


# speedup vs baseline: 1.1525x; 1.1525x over previous
import functools

import jax
import jax.numpy as jnp
from jax import lax
from jax.experimental import pallas as pl
from jax.experimental.pallas import tpu as pltpu
from jax.experimental.pallas import tpu_sc as plsc

N = 10000
E = 320000
D = 128
OUT_DIM = 64
NB = 16

NC = 2
NS = 16
EPT = E // (NC * NS)
C = 40
NCHUNK = EPT // C
CZ = 200
NCZ = N // CZ
NBUF = 4


def _sc_conv(h, src, dst, attr, We, be):
    mesh = plsc.VectorSubcoreMesh(core_axis_name="c", subcore_axis_name="s")

    @functools.partial(
        pl.kernel,
        mesh=mesh,
        out_type=jax.ShapeDtypeStruct((NC, N, D), jnp.float32),
        scratch_types=[
            pltpu.VMEM((NBUF, C), jnp.int32),
            pltpu.VMEM((NBUF, C), jnp.int32),
            [pltpu.VMEM((C, D), jnp.float32) for _ in range(NBUF)],
            [pltpu.VMEM((C * 4,), jnp.float32) for _ in range(NBUF)],
            pltpu.VMEM((4, D), jnp.float32),
            pltpu.VMEM((D,), jnp.float32),
            pltpu.VMEM_SHARED((N, D), jnp.float32),
            [pltpu.SemaphoreType.DMA for _ in range(NBUF)],
            [pltpu.SemaphoreType.DMA for _ in range(NBUF)],
            [pltpu.SemaphoreType.DMA for _ in range(NBUF)],
            [pltpu.SemaphoreType.DMA for _ in range(NBUF)],
            [pltpu.SemaphoreType.DMA for _ in range(NBUF)],
        ],
    )
    def body(h_hbm, src_hbm, dst_hbm, attr_hbm, we_hbm, be_hbm, out_hbm,
             idx_s, idx_d, rows, abuf, we_v, be_v, accum,
             isem, dsem, gsem, asem, ssem):
        cid = lax.axis_index("c")
        sid = lax.axis_index("s")
        ebase = (sid * NC + cid) * EPT

        pltpu.sync_copy(we_hbm, we_v)
        pltpu.sync_copy(be_hbm, be_v)

        def zrow(r, _):
            for c8 in range(D // 16):
                rows[NBUF - 1][r, pl.ds(c8 * 16, 16)] = (
                    jnp.zeros((16,), jnp.float32))
            return 0
        lax.fori_loop(0, C, zrow, 0)
        for t in range(N // C // NS + 1):
            zj = sid + NS * t

            @pl.when(zj < N // C)
            def _z():
                pltpu.sync_copy(rows[NBUF - 1], accum.at[pl.ds(zj * C, C)])

        def issue_idx(j, b):
            pltpu.async_copy(src_hbm.at[pl.ds(ebase + j * C, C)],
                             idx_s.at[b], isem[b])
            pltpu.async_copy(dst_hbm.at[pl.ds(ebase + j * C, C)],
                             idx_d.at[b], dsem[b])

        def issue_gather(j, b):
            pltpu.make_async_copy(src_hbm.at[pl.ds(0, C)], idx_s.at[b],
                                  isem[b]).wait()
            pltpu.async_copy(h_hbm.at[idx_s.at[b]], rows[b], gsem[b])
            pltpu.async_copy(attr_hbm.at[pl.ds((ebase + j * C) * 4, C * 4)],
                             abuf[b], asem[b])

        def wait_scatter(b):
            pltpu.make_async_copy(rows[b], accum.at[pl.ds(0, C)],
                                  ssem[b]).wait()

        def compute(j, b):
            pltpu.make_async_copy(h_hbm.at[pl.ds(0, C)], rows[b],
                                  gsem[b]).wait()
            pltpu.make_async_copy(attr_hbm.at[pl.ds(0, C * 4)], abuf[b],
                                  asem[b]).wait()

            for p in range(D // 32):
                sa, sb = pl.ds(2 * p * 16, 16), pl.ds((2 * p + 1) * 16, 16)
                wa = [we_v[k, sa] for k in range(4)]
                wb = [we_v[k, sb] for k in range(4)]
                ba = be_v[sa]
                bb = be_v[sb]

                @plsc.parallel_loop(0, C // 4, 1, unroll=1)
                def _grp(g):
                    av = abuf[b][pl.ds(g * 16, 16)]
                    for i in range(4):
                        r = g * 4 + i
                        sp = [jnp.full((16,), av[4 * i + k]) for k in range(4)]
                        acca = rows[b][r, sa] + ba
                        accb = rows[b][r, sb] + bb
                        for k in range(4):
                            acca = acca + sp[k] * wa[k]
                            accb = accb + sp[k] * wb[k]
                        rows[b][r, sa] = jnp.maximum(acca, 0.0)
                        rows[b][r, sb] = jnp.maximum(accb, 0.0)

            pltpu.make_async_copy(dst_hbm.at[pl.ds(0, C)], idx_d.at[b],
                                  dsem[b]).wait()
            pltpu.async_copy(rows[b], accum.at[idx_d.at[b]], ssem[b],
                             add=True)

        def step(j, b, issue_g=True, wait_s=True, issue_i=True,
                 guard_issue_i=False):
            if issue_g:
                issue_gather(j + 1, (b + 1) % NBUF)
            compute(j, b)
            b3 = (b + 3) % NBUF
            if wait_s:
                wait_scatter(b3)
            if issue_i:
                if guard_issue_i:
                    @pl.when(j + 3 < NCHUNK)
                    def _gi():
                        issue_idx(j + 3, b3)
                else:
                    issue_idx(j + 3, b3)

        issue_idx(0, 0)
        issue_idx(1, 1)
        issue_idx(2, 2)
        issue_gather(0, 0)
        plsc.subcore_barrier()

        step(0, 0, wait_s=False)
        step(1, 1)
        step(2, 2)
        step(3, 3)

        def group(g, _):
            for b in range(NBUF):
                step(NBUF * g + b, b, guard_issue_i=True)
            return 0
        lax.fori_loop(1, (NCHUNK - 2) // NBUF, group, 0)

        step(NCHUNK - 2, (NCHUNK - 2) % NBUF, issue_i=False)
        step(NCHUNK - 1, (NCHUNK - 1) % NBUF, issue_g=False, issue_i=False)
        wait_scatter((NCHUNK - 1) % NBUF)
        plsc.subcore_barrier()

        for t in range((NCZ + NS - 1) // NS):
            oj = sid + NS * t

            @pl.when(oj < NCZ)
            def _w():
                pltpu.sync_copy(accum.at[pl.ds(oj * CZ, CZ)],
                                out_hbm.at[cid, pl.ds(oj * CZ, CZ)])

    return body(h, src, dst, attr.reshape(E * 4), We, be)


_RB = 1000


def _linin_body(x_ref, w1_ref, b1_ref, w2_ref, b2_ref, o_ref):
    h = jnp.maximum(
        jnp.dot(x_ref[...], w1_ref[...], preferred_element_type=jnp.float32)
        + b1_ref[...], 0.0)
    o_ref[...] = jnp.maximum(
        jnp.dot(h, w2_ref[...], preferred_element_type=jnp.float32)
        + b2_ref[...], 0.0)


def _linin(x, W1, b1, W2, b2):
    grid = N // _RB
    return pl.pallas_call(
        _linin_body,
        grid=(grid,),
        in_specs=[
            pl.BlockSpec((_RB, D), lambda i: (i, 0)),
            pl.BlockSpec((D, D), lambda i: (0, 0)),
            pl.BlockSpec((1, D), lambda i: (0, 0)),
            pl.BlockSpec((D, D), lambda i: (0, 0)),
            pl.BlockSpec((1, D), lambda i: (0, 0)),
        ],
        out_specs=pl.BlockSpec((_RB, D), lambda i: (i, 0)),
        out_shape=jax.ShapeDtypeStruct((N, D), jnp.float32),
    )(x, W1, b1.reshape(1, D), W2, b2.reshape(1, D))


def _post_body(p_ref, w_ref, b_ref, o_ref):
    agg = p_ref[0] + p_ref[1]
    o_ref[...] = jnp.maximum(
        jnp.dot(agg, w_ref[...], preferred_element_type=jnp.float32)
        + b_ref[...], 0.0)


def _post(p, Wn, bn):
    grid = N // _RB
    return pl.pallas_call(
        _post_body,
        grid=(grid,),
        in_specs=[
            pl.BlockSpec((NC, _RB, D), lambda i: (0, i, 0)),
            pl.BlockSpec((D, D), lambda i: (0, 0)),
            pl.BlockSpec((1, D), lambda i: (0, 0)),
        ],
        out_specs=pl.BlockSpec((_RB, D), lambda i: (i, 0)),
        out_shape=jax.ShapeDtypeStruct((N, D), jnp.float32),
    )(p, Wn, bn.reshape(1, D))


def _final_body(p_ref, w_ref, b_ref, bt_ref, wp_ref, bp_ref, o_ref,
                sums_ref, cnts_ref):
    i = pl.program_id(0)

    @pl.when(i == 0)
    def _init():
        sums_ref[...] = jnp.zeros_like(sums_ref)
        cnts_ref[...] = jnp.zeros_like(cnts_ref)

    agg = p_ref[0] + p_ref[1]
    h2 = jnp.maximum(
        jnp.dot(agg, w_ref[...], preferred_element_type=jnp.float32)
        + b_ref[...], 0.0)
    bt = bt_ref[0]
    onehot = (jnp.broadcast_to(bt, (NB, _RB))
              == lax.broadcasted_iota(jnp.int32, (NB, _RB), 0)
              ).astype(jnp.float32)
    sums_ref[...] += lax.dot_general(
        onehot, h2, (((1,), (0,)), ((), ())),
        preferred_element_type=jnp.float32)
    cnts_ref[...] += lax.dot_general(
        onehot, jnp.ones((_RB, D), jnp.float32), (((1,), (0,)), ((), ())),
        preferred_element_type=jnp.float32)

    @pl.when(i == pl.num_programs(0) - 1)
    def _fin():
        hg = sums_ref[...] / jnp.maximum(cnts_ref[...], 1.0)
        o_ref[...] = (jnp.dot(hg, wp_ref[...],
                              preferred_element_type=jnp.float32)
                      + bp_ref[...])


def _final(p, Wn, bn, batch3d, Wp, bp):
    grid = N // _RB
    return pl.pallas_call(
        _final_body,
        grid=(grid,),
        in_specs=[
            pl.BlockSpec((NC, _RB, D), lambda i: (0, i, 0)),
            pl.BlockSpec((D, D), lambda i: (0, 0)),
            pl.BlockSpec((1, D), lambda i: (0, 0)),
            pl.BlockSpec((1, 1, _RB), lambda i: (i, 0, 0)),
            pl.BlockSpec((D, OUT_DIM), lambda i: (0, 0)),
            pl.BlockSpec((1, OUT_DIM), lambda i: (0, 0)),
        ],
        out_specs=pl.BlockSpec((NB, OUT_DIM), lambda i: (0, 0)),
        out_shape=jax.ShapeDtypeStruct((NB, OUT_DIM), jnp.float32),
        scratch_shapes=[
            pltpu.VMEM((NB, D), jnp.float32),
            pltpu.VMEM((NB, D), jnp.float32),
        ],
    )(p, Wn, bn.reshape(1, D), batch3d, Wp, bp.reshape(1, OUT_DIM))


def kernel(x, edge_index, edge_attr, batch, W_in1, b_in1, W_in2, b_in2,
           We0, be0, Wn0, bn0, We1, be1, Wn1, bn1, Wp, bp):
    src = edge_index[0]
    dst = edge_index[1]
    batch3d = batch.reshape(N // _RB, 1, _RB)

    h = _linin(x, W_in1, b_in1, W_in2, b_in2)
    p = _sc_conv(h, src, dst, edge_attr, We0, be0)
    h = _post(p, Wn0, bn0)
    p = _sc_conv(h, src, dst, edge_attr, We1, be1)
    return _final(p, Wn1, bn1, batch3d, Wp, bp)
